# baseline (speedup 1.0000x reference)
import functools

import jax
import jax.numpy as jnp
from jax import lax
from jax.experimental import pallas as pl
from jax.experimental.pallas import tpu as pltpu
from jax.experimental.pallas import tpu_sc as plsc

NC = 2
NS = 16
NW = NC * NS
CHUNK = 128

_MESH = plsc.VectorSubcoreMesh(core_axis_name="c", subcore_axis_name="s")


def _hist_kernel(n2, nchunk, stripe):
    @functools.partial(
        pl.kernel,
        out_type=jax.ShapeDtypeStruct((NC * n2,), jnp.float32),
        mesh=_MESH,
        scratch_types=[
            pltpu.VMEM((nchunk, CHUNK), jnp.int32),
            pltpu.VMEM((CHUNK,), jnp.float32),
            pltpu.VMEM((stripe,), jnp.float32),
            pltpu.VMEM_SHARED((n2,), jnp.float32),
        ],
    )
    def hist(dst_hbm, ones_hbm, zeros_hbm, cnt_hbm, idx_v, ones_v, stage_v,
             cnt_sh):
        c = lax.axis_index("c")
        s = lax.axis_index("s")
        wid = c * NS + s
        pltpu.sync_copy(zeros_hbm, stage_v)
        pltpu.sync_copy(stage_v, cnt_sh.at[pl.ds(s * stripe, stripe)])
        pltpu.sync_copy(ones_hbm, ones_v)
        pltpu.sync_copy(dst_hbm.at[wid], idx_v)
        plsc.subcore_barrier()

        def body(j, carry):
            pltpu.sync_copy(ones_v, cnt_sh.at[idx_v.at[j]], add=True)
            return carry

        lax.fori_loop(0, nchunk, body, 0)
        plsc.subcore_barrier()
        pltpu.sync_copy(cnt_sh.at[pl.ds(s * stripe, stripe)], stage_v)
        pltpu.sync_copy(stage_v, cnt_hbm.at[pl.ds(c * n2 + s * stripe, stripe)])

    return hist


def _scatter_kernel(n2, nchunk, stripe, dh):
    nq = 4
    qch = nchunk // nq
    nbuf = 4

    @functools.partial(
        pl.kernel,
        out_type=jax.ShapeDtypeStruct((n2, 2 * dh), jnp.float32),
        mesh=_MESH,
        scratch_types=[
            pltpu.VMEM((qch + nbuf, CHUNK), jnp.int32),
            pltpu.VMEM((qch, CHUNK), jnp.int32),
            pltpu.VMEM((nbuf, CHUNK, dh), jnp.float32),
            pltpu.VMEM((dh,), jnp.float32),
            pltpu.VMEM_SHARED((n2, dh), jnp.float32),
            pltpu.VMEM_SHARED((n2, dh), jnp.float32),
            pltpu.SemaphoreType.DMA,
            pltpu.SemaphoreType.DMA,
            pltpu.SemaphoreType.DMA,
            pltpu.SemaphoreType.DMA,
            pltpu.SemaphoreType.DMA,
            pltpu.SemaphoreType.DMA,
            pltpu.SemaphoreType.DMA,
            pltpu.SemaphoreType.DMA,
        ],
        compiler_params=pltpu.CompilerParams(use_tc_tiling_on_sc=False),
    )
    def scat(src_hbm, dst_hbm, g_hbm, disb_hbm, b_hbm, zeros_hbm, out_hbm,
             si_v, di_v, bufs_v, bbuf_v, g_sh, acc_sh,
             g0, g1, g2, g3, s0, s1, s2, s3):
        gsems = (g0, g1, g2, g3)
        ssems = (s0, s1, s2, s3)
        c = lax.axis_index("c")
        s = lax.axis_index("s")
        npc = stripe // CHUNK
        pltpu.sync_copy(zeros_hbm, bufs_v.at[0])
        for k in range(npc):
            pltpu.sync_copy(
                bufs_v.at[0], acc_sh.at[pl.ds(s * stripe + k * CHUNK, CHUNK)])
        for k in range(npc):
            sl = pl.ds(s * stripe + k * CHUNK, CHUNK)
            pltpu.sync_copy(g_hbm.at[c, sl], bufs_v.at[0])
            pltpu.sync_copy(bufs_v.at[0], g_sh.at[sl])
        pltpu.sync_copy(b_hbm.at[c], bbuf_v)
        zero16 = jnp.zeros((16,), jnp.int32)
        for r in range(nbuf):
            for k in range(CHUNK // 16):
                si_v[qch + r, pl.ds(k * 16, 16)] = zero16
        plsc.subcore_barrier()

        for q in range(nq):
            pltpu.sync_copy(
                src_hbm.at[s, pl.ds(q * qch, qch)], si_v.at[pl.ds(0, qch)])
            pltpu.sync_copy(dst_hbm.at[s, pl.ds(q * qch, qch)], di_v)
            for k in range(nbuf):
                pltpu.async_copy(g_sh.at[si_v.at[k]], bufs_v.at[k], gsems[k])

            def body(t, carry):
                for k in range(nbuf):
                    pltpu.make_async_copy(
                        g_sh.at[si_v.at[0]], bufs_v.at[k], gsems[k]).wait()
                    pltpu.async_copy(
                        bufs_v.at[k], acc_sh.at[di_v.at[t * nbuf + k]],
                        ssems[k], add=True)
                for k in range(nbuf):
                    pltpu.make_async_copy(
                        bufs_v.at[k], acc_sh.at[di_v.at[0]], ssems[k]).wait()
                    pltpu.async_copy(
                        g_sh.at[si_v.at[(t + 1) * nbuf + k]], bufs_v.at[k],
                        gsems[k])
                return carry

            lax.fori_loop(0, qch // nbuf, body, 0)
            for k in range(nbuf):
                pltpu.make_async_copy(
                    g_sh.at[si_v.at[0]], bufs_v.at[k], gsems[k]).wait()
        plsc.subcore_barrier()
        for k in range(npc):
            sl = pl.ds(s * stripe + k * CHUNK, CHUNK)
            pltpu.sync_copy(acc_sh.at[sl], bufs_v.at[0])
            pltpu.sync_copy(g_sh.at[sl], bufs_v.at[1])
            pltpu.sync_copy(disb_hbm.at[sl], bufs_v.at[2])

            def fin(r, carry):
                for kk in range(dh // 16):
                    s16 = pl.ds(kk * 16, 16)
                    bufs_v[0, r, s16] = (
                        bufs_v[2, r, s16]
                        * (bufs_v[0, r, s16] + bufs_v[1, r, s16])
                        + bbuf_v[s16])
                return carry

            lax.fori_loop(0, CHUNK, fin, 0)
            pltpu.sync_copy(bufs_v.at[0], out_hbm.at[sl, pl.ds(c * dh, dh)])

    return scat


def _mm_body(x_ref, w_ref, c0_ref, c1_ref, g_ref, d_ref):
    deg = c0_ref[...] + c1_ref[...] + 1.0
    dis = lax.rsqrt(deg)
    h = jnp.dot(x_ref[...], w_ref[0], preferred_element_type=jnp.float32)
    g_ref[0] = h * dis
    d_ref[...] = jnp.broadcast_to(dis, d_ref.shape)


def kernel(x, edge_index, W, b):
    n, d_in = x.shape
    d_out = W.shape[1]
    dh = d_out // 2
    e = edge_index.shape[1]

    stripe = -(-n // (NS * CHUNK)) * CHUNK
    n2 = stripe * NS
    epw_h = -(-e // (NW * CHUNK)) * CHUNK
    nch_h = epw_h // CHUNK
    e2_h = epw_h * NW
    epw_s = -(-e // (NS * 16 * CHUNK)) * 16 * CHUNK
    nch_s = epw_s // CHUNK
    e2_s = epw_s * NS

    src = edge_index[0].astype(jnp.int32)
    dst = edge_index[1].astype(jnp.int32)
    dst_h = jnp.concatenate(
        [dst, jnp.full((e2_h - e,), n, jnp.int32)]).reshape(NW, nch_h, CHUNK)
    src_s = jnp.concatenate(
        [src, jnp.zeros((e2_s - e,), jnp.int32)]).reshape(NS, nch_s, CHUNK)
    dst_s = jnp.concatenate(
        [dst, jnp.full((e2_s - e,), n, jnp.int32)]).reshape(NS, nch_s, CHUNK)

    ones_c = jnp.ones((CHUNK,), jnp.float32)
    zeros_s = jnp.zeros((stripe,), jnp.float32)
    zeros_cd = jnp.zeros((CHUNK, dh), jnp.float32)

    cnt = _hist_kernel(n2, nch_h, stripe)(dst_h, ones_c, zeros_s)
    cnt = cnt.reshape(NC, n2)
    c0 = cnt[0, :n].reshape(n, 1)
    c1 = cnt[1, :n].reshape(n, 1)

    blk = 1000
    grid = n // blk
    g2, disb = pl.pallas_call(
        _mm_body,
        grid=(NC, grid),
        in_specs=[
            pl.BlockSpec((blk, d_in), lambda j, i: (i, 0)),
            pl.BlockSpec((1, d_in, dh), lambda j, i: (j, 0, 0)),
            pl.BlockSpec((blk, 1), lambda j, i: (i, 0)),
            pl.BlockSpec((blk, 1), lambda j, i: (i, 0)),
        ],
        out_specs=[
            pl.BlockSpec((1, blk, dh), lambda j, i: (j, i, 0)),
            pl.BlockSpec((blk, dh), lambda j, i: (i, 0)),
        ],
        out_shape=[
            jax.ShapeDtypeStruct((NC, n2, dh), jnp.float32),
            jax.ShapeDtypeStruct((n2, dh), jnp.float32),
        ],
    )(x, jnp.moveaxis(W.reshape(d_in, NC, dh), 1, 0), c0, c1)

    out = _scatter_kernel(n2, nch_s, stripe, dh)(
        src_s, dst_s, g2, disb, b.reshape(NC, dh), zeros_cd)
    return out[:n]


# speedup vs baseline: 1.1810x; 1.1810x over previous
import functools

import jax
import jax.numpy as jnp
from jax import lax
from jax.experimental import pallas as pl
from jax.experimental.pallas import tpu as pltpu
from jax.experimental.pallas import tpu_sc as plsc

NC = 2
NS = 16
NW = NC * NS
CHUNK = 128

_MESH = plsc.VectorSubcoreMesh(core_axis_name="c", subcore_axis_name="s")


def _hist_kernel(n2, nchunk, stripe):
    @functools.partial(
        pl.kernel,
        out_type=jax.ShapeDtypeStruct((NC * n2,), jnp.float32),
        mesh=_MESH,
        scratch_types=[
            pltpu.VMEM((nchunk, CHUNK), jnp.int32),
            pltpu.VMEM((CHUNK,), jnp.float32),
            pltpu.VMEM((stripe,), jnp.float32),
            pltpu.VMEM_SHARED((n2,), jnp.float32),
        ],
    )
    def hist(dst_hbm, ones_hbm, zeros_hbm, cnt_hbm, idx_v, ones_v, stage_v,
             cnt_sh):
        c = lax.axis_index("c")
        s = lax.axis_index("s")
        wid = c * NS + s
        pltpu.sync_copy(zeros_hbm, stage_v)
        pltpu.sync_copy(stage_v, cnt_sh.at[pl.ds(s * stripe, stripe)])
        pltpu.sync_copy(ones_hbm, ones_v)
        pltpu.sync_copy(dst_hbm.at[wid], idx_v)
        plsc.subcore_barrier()

        def body(j, carry):
            pltpu.sync_copy(ones_v, cnt_sh.at[idx_v.at[j]], add=True)
            return carry

        lax.fori_loop(0, nchunk, body, 0)
        plsc.subcore_barrier()
        pltpu.sync_copy(cnt_sh.at[pl.ds(s * stripe, stripe)], stage_v)
        pltpu.sync_copy(stage_v, cnt_hbm.at[pl.ds(c * n2 + s * stripe, stripe)])

    return hist


def _scatter_kernel(n2, nchunk, stripe, dh):
    nhalf = nchunk // 2

    @functools.partial(
        pl.kernel,
        out_type=jax.ShapeDtypeStruct((n2, 2 * dh), jnp.float32),
        mesh=_MESH,
        scratch_types=[
            pltpu.VMEM((nhalf + 1, CHUNK), jnp.int32),
            pltpu.VMEM((nhalf, CHUNK), jnp.int32),
            pltpu.VMEM((CHUNK, dh), jnp.float32),
            pltpu.VMEM((CHUNK, dh), jnp.float32),
            pltpu.VMEM((CHUNK, dh), jnp.float32),
            pltpu.VMEM((dh,), jnp.float32),
            pltpu.VMEM_SHARED((n2, dh), jnp.float32),
            pltpu.VMEM_SHARED((n2, dh), jnp.float32),
            pltpu.SemaphoreType.DMA,
            pltpu.SemaphoreType.DMA,
        ],
        compiler_params=pltpu.CompilerParams(use_tc_tiling_on_sc=False),
    )
    def scat(src_hbm, dst_hbm, g_hbm, disb_hbm, b_hbm, zeros_hbm, out_hbm,
             si_v, di_v, rows_v, rowsb_v, dbuf_v, bbuf_v,
             g_sh, acc_sh, sem, semb):
        c = lax.axis_index("c")
        s = lax.axis_index("s")
        npc = stripe // CHUNK
        pltpu.sync_copy(zeros_hbm, rows_v)
        for k in range(npc):
            pltpu.sync_copy(
                rows_v, acc_sh.at[pl.ds(s * stripe + k * CHUNK, CHUNK)])
        for k in range(npc):
            sl = pl.ds(s * stripe + k * CHUNK, CHUNK)
            pltpu.sync_copy(g_hbm.at[c, sl], rows_v)
            pltpu.sync_copy(rows_v, g_sh.at[sl])
        pltpu.sync_copy(b_hbm.at[c], bbuf_v)
        zero16 = jnp.zeros((16,), jnp.int32)
        for k in range(CHUNK // 16):
            si_v[nhalf, pl.ds(k * 16, 16)] = zero16
        plsc.subcore_barrier()

        for h in range(2):
            pltpu.sync_copy(
                src_hbm.at[s, pl.ds(h * nhalf, nhalf)],
                si_v.at[pl.ds(0, nhalf)])
            pltpu.sync_copy(dst_hbm.at[s, pl.ds(h * nhalf, nhalf)], di_v)
            pltpu.async_copy(g_sh.at[si_v.at[0]], rows_v, sem)

            def body(jj, carry):
                j0 = jj * 2
                pltpu.make_async_copy(
                    g_sh.at[si_v.at[0]], rows_v, sem).wait()
                pltpu.async_copy(g_sh.at[si_v.at[j0 + 1]], rowsb_v, semb)
                pltpu.sync_copy(rows_v, acc_sh.at[di_v.at[j0]], add=True)
                pltpu.make_async_copy(
                    g_sh.at[si_v.at[0]], rowsb_v, semb).wait()
                pltpu.async_copy(g_sh.at[si_v.at[j0 + 2]], rows_v, sem)
                pltpu.sync_copy(rowsb_v, acc_sh.at[di_v.at[j0 + 1]], add=True)
                return carry

            lax.fori_loop(0, nhalf // 2, body, 0)
            pltpu.make_async_copy(g_sh.at[si_v.at[0]], rows_v, sem).wait()
        plsc.subcore_barrier()
        for k in range(npc):
            sl = pl.ds(s * stripe + k * CHUNK, CHUNK)
            pltpu.sync_copy(acc_sh.at[sl], rows_v)
            pltpu.sync_copy(g_sh.at[sl], rowsb_v)
            pltpu.sync_copy(disb_hbm.at[sl], dbuf_v)

            def fin(r, carry):
                for kk in range(dh // 16):
                    s16 = pl.ds(kk * 16, 16)
                    rows_v[r, s16] = (
                        dbuf_v[r, s16] * (rows_v[r, s16] + rowsb_v[r, s16])
                        + bbuf_v[s16])
                return carry

            lax.fori_loop(0, CHUNK, fin, 0)
            pltpu.sync_copy(rows_v, out_hbm.at[sl, pl.ds(c * dh, dh)])

    return scat


def _mm_body(x_ref, w_ref, c0_ref, c1_ref, g_ref, d_ref):
    deg = c0_ref[...] + c1_ref[...] + 1.0
    dis = lax.rsqrt(deg)
    h = jnp.dot(x_ref[...], w_ref[0], preferred_element_type=jnp.float32)
    g_ref[0] = h * dis
    d_ref[...] = jnp.broadcast_to(dis, d_ref.shape)


def kernel(x, edge_index, W, b):
    n, d_in = x.shape
    d_out = W.shape[1]
    dh = d_out // 2
    e = edge_index.shape[1]

    stripe = -(-n // (NS * CHUNK)) * CHUNK
    n2 = stripe * NS
    epw_h = -(-e // (NW * CHUNK)) * CHUNK
    nch_h = epw_h // CHUNK
    e2_h = epw_h * NW
    epw_s = -(-e // (NS * 4 * CHUNK)) * 4 * CHUNK
    nch_s = epw_s // CHUNK
    e2_s = epw_s * NS

    src = edge_index[0].astype(jnp.int32)
    dst = edge_index[1].astype(jnp.int32)
    dst_h = jnp.concatenate(
        [dst, jnp.full((e2_h - e,), n, jnp.int32)]).reshape(NW, nch_h, CHUNK)
    src_s = jnp.concatenate(
        [src, jnp.zeros((e2_s - e,), jnp.int32)]).reshape(NS, nch_s, CHUNK)
    dst_s = jnp.concatenate(
        [dst, jnp.full((e2_s - e,), n, jnp.int32)]).reshape(NS, nch_s, CHUNK)

    ones_c = jnp.ones((CHUNK,), jnp.float32)
    zeros_s = jnp.zeros((stripe,), jnp.float32)
    zeros_cd = jnp.zeros((CHUNK, dh), jnp.float32)

    cnt = _hist_kernel(n2, nch_h, stripe)(dst_h, ones_c, zeros_s)
    cnt = cnt.reshape(NC, n2)
    c0 = cnt[0, :n].reshape(n, 1)
    c1 = cnt[1, :n].reshape(n, 1)

    blk = 2000
    grid = n // blk
    g2, disb = pl.pallas_call(
        _mm_body,
        grid=(grid, NC),
        in_specs=[
            pl.BlockSpec((blk, d_in), lambda i, j: (i, 0)),
            pl.BlockSpec((1, d_in, dh), lambda i, j: (j, 0, 0)),
            pl.BlockSpec((blk, 1), lambda i, j: (i, 0)),
            pl.BlockSpec((blk, 1), lambda i, j: (i, 0)),
        ],
        out_specs=[
            pl.BlockSpec((1, blk, dh), lambda i, j: (j, i, 0)),
            pl.BlockSpec((blk, dh), lambda i, j: (i, 0)),
        ],
        out_shape=[
            jax.ShapeDtypeStruct((NC, n2, dh), jnp.float32),
            jax.ShapeDtypeStruct((n2, dh), jnp.float32),
        ],
    )(x, jnp.moveaxis(W.reshape(d_in, NC, dh), 1, 0), c0, c1)

    out = _scatter_kernel(n2, nch_s, stripe, dh)(
        src_s, dst_s, g2, disb, b.reshape(NC, dh), zeros_cd)
    return out[:n]
